# baseline (speedup 1.0000x reference)
import jax
import jax.numpy as jnp
from jax import lax
from jax.experimental import pallas as pl
from jax.experimental.pallas import tpu as pltpu
from jax.experimental.pallas import tpu_sc as plsc

_NC = 2
_NS = 16
_NW = _NC * _NS
_LANES = 16

_B = 4096
_E = 64
_H = 50
_TBLK = 512
_NPBLK = 977
_D = _NPBLK * _TBLK
_B_PER_W = _B // _NW
_CHUNKS = _B_PER_W
_NBUF = 4
_QS = _E // _LANES


def _gather_sum_body(idx_hbm, table_hbm, out_hbm,
                     raw_v, idx_v, rows_v, outb_v, s0, s1, s2, s3):
    sems = (s0, s1, s2, s3)
    wid = lax.axis_index("s") * _NC + lax.axis_index("c")
    base = pl.multiple_of(wid * _B_PER_W, 8)

    pltpu.sync_copy(idx_hbm.at[pl.ds(base, _B_PER_W)], raw_v)

    def conv(r, carry):
        for col in (0, 16, 32, _H - _LANES):
            v = raw_v[r, pl.ds(col, _LANES)]
            idx_v[r, pl.ds(col, _LANES)] = jnp.where(
                v < _D, v + v, v + v - jnp.int32(2 * _D - 1)
            )
        return carry

    plsc.parallel_loop(0, _B_PER_W, unroll=4, carry=jnp.int32(0))(conv)

    def gather_start(c, b):
        pltpu.async_copy(table_hbm.at[idx_v.at[c]], rows_v.at[b], sems[b])

    def gather_wait(c, b):
        pltpu.make_async_copy(
            table_hbm.at[idx_v.at[c]], rows_v.at[b], sems[b]
        ).wait()

    for b in range(_NBUF):
        gather_start(b, b)

    def reduce_rows(rb):
        zero = jnp.zeros((_LANES,), jnp.float32)
        init = (zero, zero, zero, zero)

        def red(j, acc):
            return tuple(
                acc[q] + rb[j, pl.ds(q * _LANES, _LANES)] for q in range(_QS)
            )

        return plsc.parallel_loop(0, _H, unroll=10, carry=init)(red)

    def t_body(t, carry):
        for b in range(_NBUF):
            c = t * _NBUF + b
            gather_wait(c, b)
            acc = reduce_rows(rows_v.at[b])
            for q in range(_QS):
                outb_v[c, pl.ds(q * _LANES, _LANES)] = acc[q]
            nc = c + _NBUF

            @pl.when(nc < _CHUNKS)
            def _():
                gather_start(nc, b)

        return carry

    lax.fori_loop(0, _CHUNKS // _NBUF, t_body, 0)

    pltpu.sync_copy(outb_v, out_hbm.at[pl.ds(base, _B_PER_W)])


def _gather_sum(idx, table2):
    k = pl.kernel(
        _gather_sum_body,
        out_type=jax.ShapeDtypeStruct((_B, _E), jnp.float32),
        mesh=plsc.VectorSubcoreMesh(
            core_axis_name="c", subcore_axis_name="s",
            num_cores=_NC, num_subcores=_NS,
        ),
        scratch_types=[
            pltpu.VMEM((_B_PER_W, _H), jnp.int32),
            pltpu.VMEM((_B_PER_W, _H), jnp.int32),
            pltpu.VMEM((_NBUF, _H, _E), jnp.float32),
            pltpu.VMEM((_B_PER_W, _E), jnp.float32),
            pltpu.SemaphoreType.DMA,
            pltpu.SemaphoreType.DMA,
            pltpu.SemaphoreType.DMA,
            pltpu.SemaphoreType.DMA,
        ],
        compiler_params=pltpu.CompilerParams(use_tc_tiling_on_sc=False),
    )
    return k(idx, table2)


def _packT_body(a_ref, b_ref, o_ref):
    o_ref[:, 0:_E] = a_ref[...].T
    o_ref[:, _E:2 * _E] = b_ref[...].T


def _pack_transpose(tt):
    return pl.pallas_call(
        _packT_body,
        grid=(_NPBLK,),
        in_specs=[
            pl.BlockSpec((_E, _TBLK), lambda i: (0, i)),
            pl.BlockSpec((_E, _TBLK), lambda i: (0, i + _NPBLK)),
        ],
        out_specs=pl.BlockSpec((_TBLK, 2 * _E), lambda i: (i, 0)),
        out_shape=jax.ShapeDtypeStruct((_D, 2 * _E), jnp.float32),
    )(tt, tt)


def _proj_body(x_ref, w_ref, b_ref, o_ref):
    x = jnp.maximum(x_ref[...], 0.0)
    o_ref[...] = (
        lax.dot_general(
            w_ref[...], x,
            dimension_numbers=(((1,), (1,)), ((), ())),
            preferred_element_type=jnp.float32,
        )
        + b_ref[...]
    )


def _proj_t(x, W, bcol):
    B, E = x.shape
    T = W.shape[0]
    blk = 512
    return pl.pallas_call(
        _proj_body,
        grid=(B // blk,),
        in_specs=[
            pl.BlockSpec((blk, E), lambda i: (i, 0)),
            pl.BlockSpec((T, E), lambda i: (0, 0)),
            pl.BlockSpec((T, 1), lambda i: (0, 0)),
        ],
        out_specs=pl.BlockSpec((T, blk), lambda i: (0, i)),
        out_shape=jax.ShapeDtypeStruct((T, B), jnp.float32),
    )(x, W, bcol)


def kernel(input_text, table, W, b):
    table2 = _pack_transpose(table.T).reshape(2 * _D, _E)
    sums = _gather_sum(input_text, table2)
    out_t = _proj_t(sums, W, b.reshape(-1, 1))
    return out_t.T


# speedup vs baseline: 1.1259x; 1.1259x over previous
import jax
import jax.numpy as jnp
from jax import lax
from jax.experimental import pallas as pl
from jax.experimental.pallas import tpu as pltpu
from jax.experimental.pallas import tpu_sc as plsc

_NC = 2
_NS = 16
_NW = _NC * _NS
_LANES = 16

_B = 4096
_E = 64
_H = 50
_V = 1000000
_B_PER_W = _B // _NW
_CHUNKS = _B_PER_W
_NBUF = 4
_QS = _E // _LANES

_DB = 256
_NDB = 3906
_DTAIL = _V - _NDB * _DB
_KMAX = 124


def _detile_body(table_hbm, out_hbm, buf_v, buf1_v, tb_v, tb1_v,
                 si0, si1, so0, so1):
    sin = (si0, si1)
    sout = (so0, so1)
    wid = lax.axis_index("s") * _NC + lax.axis_index("c")

    def blk_of(k):
        return jnp.minimum(wid + _NW * k, _NDB - 1)

    def valid(k):
        return (wid + _NW * k) < _NDB

    def start_in(k, bb):
        r0 = pl.multiple_of(blk_of(k) * _DB, 8)
        pltpu.async_copy(table_hbm.at[pl.ds(r0, _DB)], buf_v.at[bb], sin[bb])

    def wait_in(bb):
        pltpu.make_async_copy(
            table_hbm.at[pl.ds(0, _DB)], buf_v.at[bb], sin[bb]
        ).wait()

    def start_out(k, bb):
        o0 = pl.multiple_of(blk_of(k) * (_DB * _E), 8)
        pltpu.async_copy(buf1_v.at[bb], out_hbm.at[pl.ds(o0, _DB * _E)],
                         sout[bb])

    def wait_out(bb):
        pltpu.make_async_copy(
            buf1_v.at[bb], out_hbm.at[pl.ds(0, _DB * _E)], sout[bb]
        ).wait()

    for bb in range(2):
        start_in(bb, bb)

    def k_body(k2, carry):
        for bb in range(2):
            k = k2 * 2 + bb
            wait_in(bb)

            @pl.when(k >= 2)
            def _():
                wait_out(bb)

            def flatten(r, c):
                for q in range(_QS):
                    buf1_v[bb, pl.ds(r * _E + q * _LANES, _LANES)] = (
                        buf_v[bb, r, pl.ds(q * _LANES, _LANES)]
                    )
                return c

            plsc.parallel_loop(0, _DB, unroll=4, carry=jnp.int32(0))(flatten)

            @pl.when(valid(k))
            def _():
                start_out(k, bb)

            @pl.when(k + 2 < _KMAX)
            def _():
                start_in(k + 2, bb)

        return carry

    lax.fori_loop(0, _KMAX // 2, k_body, 0)
    for bb in range(2):
        @pl.when(valid(_KMAX - 2 + bb))
        def _():
            wait_out(bb)

    @pl.when(wid == 2)
    def _():
        r0 = _NDB * _DB
        pltpu.sync_copy(table_hbm.at[pl.ds(r0, _DTAIL)], tb_v)

        def tflat(r, c):
            for q in range(_QS):
                tb1_v[pl.ds(r * _E + q * _LANES, _LANES)] = (
                    tb_v[r, pl.ds(q * _LANES, _LANES)]
                )
            return c

        plsc.parallel_loop(0, _DTAIL, unroll=4, carry=jnp.int32(0))(tflat)
        pltpu.sync_copy(tb1_v, out_hbm.at[pl.ds(r0 * _E, _DTAIL * _E)])


def _detile(table):
    k = pl.kernel(
        _detile_body,
        out_type=jax.ShapeDtypeStruct((_V * _E,), jnp.float32),
        mesh=plsc.VectorSubcoreMesh(
            core_axis_name="c", subcore_axis_name="s",
            num_cores=_NC, num_subcores=_NS,
        ),
        scratch_types=[
            pltpu.VMEM((2, _DB, _E), jnp.float32),
            pltpu.VMEM((2, _DB * _E), jnp.float32),
            pltpu.VMEM((_DTAIL, _E), jnp.float32),
            pltpu.VMEM((_DTAIL * _E,), jnp.float32),
            pltpu.SemaphoreType.DMA,
            pltpu.SemaphoreType.DMA,
            pltpu.SemaphoreType.DMA,
            pltpu.SemaphoreType.DMA,
        ],
        compiler_params=pltpu.CompilerParams(use_tc_tiling_on_sc=True),
    )
    return k(table)


def _gather_sum_body(idx_hbm, table_hbm, out_hbm,
                     idx_v, rows_v, outb_v, s0, s1, s2, s3):
    sems = (s0, s1, s2, s3)
    wid = lax.axis_index("s") * _NC + lax.axis_index("c")
    base = pl.multiple_of(wid * _B_PER_W, 8)

    pltpu.sync_copy(idx_hbm.at[pl.ds(base, _B_PER_W)], idx_v)

    def gather_start(c, b):
        pltpu.async_copy(table_hbm.at[idx_v.at[c]], rows_v.at[b], sems[b])

    def gather_wait(c, b):
        pltpu.make_async_copy(
            table_hbm.at[idx_v.at[c]], rows_v.at[b], sems[b]
        ).wait()

    for b in range(_NBUF):
        gather_start(b, b)

    def reduce_rows(rb):
        zero = jnp.zeros((_LANES,), jnp.float32)
        init = (zero, zero, zero, zero)

        def red(j, acc):
            return tuple(
                acc[q] + rb[j, pl.ds(q * _LANES, _LANES)] for q in range(_QS)
            )

        return plsc.parallel_loop(0, _H, unroll=10, carry=init)(red)

    def t_body(t, carry):
        for b in range(_NBUF):
            c = t * _NBUF + b
            gather_wait(c, b)
            acc = reduce_rows(rows_v.at[b])
            for q in range(_QS):
                outb_v[c, pl.ds(q * _LANES, _LANES)] = acc[q]
            nc = c + _NBUF

            @pl.when(nc < _CHUNKS)
            def _():
                gather_start(nc, b)

        return carry

    lax.fori_loop(0, _CHUNKS // _NBUF, t_body, 0)

    pltpu.sync_copy(outb_v, out_hbm.at[pl.ds(base, _B_PER_W)])


def _gather_sum(idx, table2):
    k = pl.kernel(
        _gather_sum_body,
        out_type=jax.ShapeDtypeStruct((_B, _E), jnp.float32),
        mesh=plsc.VectorSubcoreMesh(
            core_axis_name="c", subcore_axis_name="s",
            num_cores=_NC, num_subcores=_NS,
        ),
        scratch_types=[
            pltpu.VMEM((_B_PER_W, _H), jnp.int32),
            pltpu.VMEM((_NBUF, _H, _E), jnp.float32),
            pltpu.VMEM((_B_PER_W, _E), jnp.float32),
            pltpu.SemaphoreType.DMA,
            pltpu.SemaphoreType.DMA,
            pltpu.SemaphoreType.DMA,
            pltpu.SemaphoreType.DMA,
        ],
        compiler_params=pltpu.CompilerParams(use_tc_tiling_on_sc=False),
    )
    return k(idx, table2)


def _proj_body(x_ref, w_ref, b_ref, o_ref):
    x = jnp.maximum(x_ref[...], 0.0)
    o_ref[...] = (
        lax.dot_general(
            w_ref[...], x,
            dimension_numbers=(((1,), (1,)), ((), ())),
            preferred_element_type=jnp.float32,
        )
        + b_ref[...]
    )


def _proj_t(x, W, bcol):
    B, E = x.shape
    T = W.shape[0]
    blk = 512
    return pl.pallas_call(
        _proj_body,
        grid=(B // blk,),
        in_specs=[
            pl.BlockSpec((blk, E), lambda i: (i, 0)),
            pl.BlockSpec((T, E), lambda i: (0, 0)),
            pl.BlockSpec((T, 1), lambda i: (0, 0)),
        ],
        out_specs=pl.BlockSpec((T, blk), lambda i: (0, i)),
        out_shape=jax.ShapeDtypeStruct((T, B), jnp.float32),
    )(x, W, bcol)


def kernel(input_text, table, W, b):
    table2 = _detile(table).reshape(_V, _E)
    sums = _gather_sum(input_text, table2)
    out_t = _proj_t(sums, W, b.reshape(-1, 1))
    return out_t.T
